# baseline (speedup 1.0000x reference)
import functools
import jax
import jax.numpy as jnp
from jax import lax
from jax.experimental import pallas as pl
from jax.experimental.pallas import tpu as pltpu
from jax.experimental.pallas import tpu_sc as plsc

N = 100000
E = 50
L = 16
NC = 2
NS = 16
NW = NC * NS
G = 8
TILE_ROWS = G * L
NSLAB = E // 8

TC_ROWS = 73728
BC = 12288
TC_GRID = TC_ROWS // BC

CH = 768
SC_ROWS = N - TC_ROWS
REM = 32
EXTRA = (SC_ROWS - REM) // 128 - NW * (CH // 128)
TILES = CH // TILE_ROWS
SPAN_A = CH + 128
SPAN_B = CH
SPAN_LAST = CH + REM


def _sc_body(xt_hbm, ae_hbm, out_hbm, xbuf, tailbuf, aexp, outbuf,
             sem0, sem1):
    wid = lax.axis_index("s") * NC + lax.axis_index("c")
    base_l = wid * CH + jnp.minimum(wid, EXTRA) * 128
    base = TC_ROWS + base_l
    zero = jnp.zeros((L,), jnp.float32)

    is_a = wid < EXTRA
    is_last = wid == NW - 1
    tail_row = pl.multiple_of(base + CH, 128)

    b128 = pl.multiple_of(base, 128)
    for q in range(CH // 128):
        pltpu.async_copy(xt_hbm.at[:, pl.ds(b128 + 128 * q, 128)],
                         xbuf.at[:, pl.ds(128 * q, 128)], sem0)
    pltpu.sync_copy(ae_hbm, aexp)

    @pl.when(is_a)
    def _():
        pltpu.async_copy(xt_hbm.at[:, pl.ds(tail_row, 128)], tailbuf, sem1)

    @pl.when(is_last)
    def _():
        for s in range(NSLAB):
            pltpu.async_copy(xt_hbm.at[pl.ds(8 * s, 8), pl.ds(tail_row, REM)],
                             tailbuf.at[pl.ds(8 * s, 8), pl.ds(0, REM)], sem1)
        pltpu.async_copy(xt_hbm.at[pl.ds(8 * NSLAB, 2), pl.ds(tail_row, REM)],
                         tailbuf.at[pl.ds(8 * NSLAB, 2), pl.ds(0, REM)], sem1)

    for q in range(CH // 128):
        pltpu.make_async_copy(xt_hbm.at[:, pl.ds(0, 128)],
                              xbuf.at[:, pl.ds(128 * q, 128)], sem0).wait()

    @pl.loop(0, TILES)
    def _(t):
        cbase = t * TILE_ROWS
        accs = [zero] * G
        for k in range(E):
            aev = aexp[pl.ds(L * k, L)]
            for j in range(G):
                accs[j] = accs[j] + xbuf[k, pl.ds(cbase + L * j, L)] * aev
        for j in range(G):
            outbuf[pl.ds(cbase + L * j, L)] = accs[j]

    @pl.when(is_a)
    def _():
        pltpu.make_async_copy(xt_hbm.at[:, pl.ds(0, 128)], tailbuf,
                              sem1).wait()

    @pl.when(is_last)
    def _():
        for s in range(NSLAB):
            pltpu.make_async_copy(
                xt_hbm.at[pl.ds(0, 8), pl.ds(0, REM)],
                tailbuf.at[pl.ds(8 * s, 8), pl.ds(0, REM)], sem1).wait()
        pltpu.make_async_copy(
            xt_hbm.at[pl.ds(0, 2), pl.ds(0, REM)],
            tailbuf.at[pl.ds(8 * NSLAB, 2), pl.ds(0, REM)], sem1).wait()

    tail_n = jnp.where(is_a, 128, jnp.where(is_last, REM, 0))

    @pl.loop(0, tail_n // L)
    def _(g):
        acc0 = zero
        acc1 = zero
        for k in range(0, E, 2):
            aev0 = aexp[pl.ds(L * k, L)]
            aev1 = aexp[pl.ds(L * (k + 1), L)]
            acc0 = acc0 + tailbuf[k, pl.ds(g * L, L)] * aev0
            acc1 = acc1 + tailbuf[k + 1, pl.ds(g * L, L)] * aev1
        outbuf[pl.ds(CH + g * L, L)] = acc0 + acc1

    @pl.when(is_a)
    def _():
        pltpu.sync_copy(outbuf.at[pl.ds(0, SPAN_A)],
                        out_hbm.at[pl.ds(base_l, SPAN_A)])

    @pl.when(jnp.logical_and(jnp.logical_not(is_a), jnp.logical_not(is_last)))
    def _():
        pltpu.sync_copy(outbuf.at[pl.ds(0, SPAN_B)],
                        out_hbm.at[pl.ds(base_l, SPAN_B)])

    @pl.when(is_last)
    def _():
        pltpu.sync_copy(outbuf.at[pl.ds(0, SPAN_LAST)],
                        out_hbm.at[pl.ds(base_l, SPAN_LAST)])


@functools.partial(
    pl.kernel,
    out_type=jax.ShapeDtypeStruct((SC_ROWS,), jnp.float32),
    mesh=plsc.VectorSubcoreMesh(core_axis_name="c", subcore_axis_name="s"),
    compiler_params=pltpu.CompilerParams(
        needs_layout_passes=False, use_tc_tiling_on_sc=True),
    scratch_types=[
        pltpu.VMEM((E, CH), jnp.float32),
        pltpu.VMEM((E, 128), jnp.float32),
        pltpu.VMEM((E * L,), jnp.float32),
        pltpu.VMEM((SPAN_A,), jnp.float32),
        pltpu.SemaphoreType.DMA,
        pltpu.SemaphoreType.DMA,
    ],
)
def _sc_matvec(xt, ae_exp, out_flat, xbuf, tailbuf, aexp, outbuf,
               sem0, sem1):
    _sc_body(xt, ae_exp, out_flat, xbuf, tailbuf, aexp, outbuf, sem0, sem1)


def _tc_body(xt_ref, ae_ref, out_ref):
    out_ref[...] = jax.lax.dot_general(
        ae_ref[...], xt_ref[...],
        dimension_numbers=(((1,), (0,)), ((), ())))


_tc_matvec = pl.pallas_call(
    _tc_body,
    grid=(TC_GRID,),
    in_specs=[
        pl.BlockSpec((E, BC), lambda i: (0, i)),
        pl.BlockSpec((1, E), lambda i: (0, 0)),
    ],
    out_specs=pl.BlockSpec((1, BC), lambda i: (0, i)),
    out_shape=jax.ShapeDtypeStruct((1, TC_ROWS), jnp.float32),
)


@jax.jit
def kernel(x, atomic_energies):
    xt = x.T
    ae_exp = jnp.broadcast_to(atomic_energies[:, None], (E, L)).reshape(E * L)
    out_sc = _sc_matvec(xt, ae_exp)
    out_tc = _tc_matvec(xt, atomic_energies[None, :])
    out = jnp.concatenate([out_tc[0], out_sc])
    return out[:, None]


# speedup vs baseline: 1.0006x; 1.0006x over previous
import functools
import jax
import jax.numpy as jnp
from jax import lax
from jax.experimental import pallas as pl
from jax.experimental.pallas import tpu as pltpu
from jax.experimental.pallas import tpu_sc as plsc

N = 100000
E = 50
L = 16
NC = 2
NS = 16
NW = NC * NS
G = 8
TILE_ROWS = G * L
NSLAB = E // 8

TC_ROWS = 77824
BC = 9728
TC_GRID = TC_ROWS // BC

CH = 640
SC_ROWS = N - TC_ROWS
REM = 32
EXTRA = (SC_ROWS - REM) // 128 - NW * (CH // 128)
TILES = CH // TILE_ROWS
SPAN_A = CH + 128
SPAN_B = CH
SPAN_LAST = CH + REM


def _sc_body(xt_hbm, ae_hbm, out_hbm, xbuf, tailbuf, aexp, outbuf,
             sem0, sem1):
    wid = lax.axis_index("s") * NC + lax.axis_index("c")
    base_l = wid * CH + jnp.minimum(wid, EXTRA) * 128
    base = TC_ROWS + base_l
    zero = jnp.zeros((L,), jnp.float32)

    is_a = wid < EXTRA
    is_last = wid == NW - 1
    tail_row = pl.multiple_of(base + CH, 128)

    b128 = pl.multiple_of(base, 128)
    for q in range(CH // 128):
        pltpu.async_copy(xt_hbm.at[:, pl.ds(b128 + 128 * q, 128)],
                         xbuf.at[:, pl.ds(128 * q, 128)], sem0)
    pltpu.sync_copy(ae_hbm, aexp)

    @pl.when(is_a)
    def _():
        pltpu.async_copy(xt_hbm.at[:, pl.ds(tail_row, 128)], tailbuf, sem1)

    @pl.when(is_last)
    def _():
        for s in range(NSLAB):
            pltpu.async_copy(xt_hbm.at[pl.ds(8 * s, 8), pl.ds(tail_row, REM)],
                             tailbuf.at[pl.ds(8 * s, 8), pl.ds(0, REM)], sem1)
        pltpu.async_copy(xt_hbm.at[pl.ds(8 * NSLAB, 2), pl.ds(tail_row, REM)],
                         tailbuf.at[pl.ds(8 * NSLAB, 2), pl.ds(0, REM)], sem1)

    for q in range(CH // 128):
        pltpu.make_async_copy(xt_hbm.at[:, pl.ds(0, 128)],
                              xbuf.at[:, pl.ds(128 * q, 128)], sem0).wait()

    @pl.loop(0, TILES)
    def _(t):
        cbase = t * TILE_ROWS
        accs = [zero] * G
        for k in range(E):
            aev = aexp[pl.ds(L * k, L)]
            for j in range(G):
                accs[j] = accs[j] + xbuf[k, pl.ds(cbase + L * j, L)] * aev
        for j in range(G):
            outbuf[pl.ds(cbase + L * j, L)] = accs[j]

    @pl.when(is_a)
    def _():
        pltpu.make_async_copy(xt_hbm.at[:, pl.ds(0, 128)], tailbuf,
                              sem1).wait()

    @pl.when(is_last)
    def _():
        for s in range(NSLAB):
            pltpu.make_async_copy(
                xt_hbm.at[pl.ds(0, 8), pl.ds(0, REM)],
                tailbuf.at[pl.ds(8 * s, 8), pl.ds(0, REM)], sem1).wait()
        pltpu.make_async_copy(
            xt_hbm.at[pl.ds(0, 2), pl.ds(0, REM)],
            tailbuf.at[pl.ds(8 * NSLAB, 2), pl.ds(0, REM)], sem1).wait()

    tail_n = jnp.where(is_a, 128, jnp.where(is_last, REM, 0))

    @pl.loop(0, tail_n // L)
    def _(g):
        acc0 = zero
        acc1 = zero
        for k in range(0, E, 2):
            aev0 = aexp[pl.ds(L * k, L)]
            aev1 = aexp[pl.ds(L * (k + 1), L)]
            acc0 = acc0 + tailbuf[k, pl.ds(g * L, L)] * aev0
            acc1 = acc1 + tailbuf[k + 1, pl.ds(g * L, L)] * aev1
        outbuf[pl.ds(CH + g * L, L)] = acc0 + acc1

    @pl.when(is_a)
    def _():
        pltpu.sync_copy(outbuf.at[pl.ds(0, SPAN_A)],
                        out_hbm.at[pl.ds(base_l, SPAN_A)])

    @pl.when(jnp.logical_and(jnp.logical_not(is_a), jnp.logical_not(is_last)))
    def _():
        pltpu.sync_copy(outbuf.at[pl.ds(0, SPAN_B)],
                        out_hbm.at[pl.ds(base_l, SPAN_B)])

    @pl.when(is_last)
    def _():
        pltpu.sync_copy(outbuf.at[pl.ds(0, SPAN_LAST)],
                        out_hbm.at[pl.ds(base_l, SPAN_LAST)])


@functools.partial(
    pl.kernel,
    out_type=jax.ShapeDtypeStruct((SC_ROWS,), jnp.float32),
    mesh=plsc.VectorSubcoreMesh(core_axis_name="c", subcore_axis_name="s"),
    compiler_params=pltpu.CompilerParams(
        needs_layout_passes=False, use_tc_tiling_on_sc=True),
    scratch_types=[
        pltpu.VMEM((E, CH), jnp.float32),
        pltpu.VMEM((E, 128), jnp.float32),
        pltpu.VMEM((E * L,), jnp.float32),
        pltpu.VMEM((SPAN_A,), jnp.float32),
        pltpu.SemaphoreType.DMA,
        pltpu.SemaphoreType.DMA,
    ],
)
def _sc_matvec(xt, ae_exp, out_flat, xbuf, tailbuf, aexp, outbuf,
               sem0, sem1):
    _sc_body(xt, ae_exp, out_flat, xbuf, tailbuf, aexp, outbuf, sem0, sem1)


def _tc_body(xt_ref, ae_ref, out_ref):
    out_ref[...] = jax.lax.dot_general(
        ae_ref[...], xt_ref[...],
        dimension_numbers=(((1,), (0,)), ((), ())))


_tc_matvec = pl.pallas_call(
    _tc_body,
    grid=(TC_GRID,),
    in_specs=[
        pl.BlockSpec((E, BC), lambda i: (0, i)),
        pl.BlockSpec((1, E), lambda i: (0, 0)),
    ],
    out_specs=pl.BlockSpec((1, BC), lambda i: (0, i)),
    out_shape=jax.ShapeDtypeStruct((1, TC_ROWS), jnp.float32),
)


@jax.jit
def kernel(x, atomic_energies):
    xt = x.T
    ae_exp = jnp.broadcast_to(atomic_energies[:, None], (E, L)).reshape(E * L)
    out_sc = _sc_matvec(xt, ae_exp)
    out_tc = _tc_matvec(xt, atomic_energies[None, :])
    out = jnp.concatenate([out_tc[0], out_sc])
    return out[:, None]


# speedup vs baseline: 1.0394x; 1.0388x over previous
import functools
import jax
import jax.numpy as jnp
from jax import lax
from jax.experimental import pallas as pl
from jax.experimental.pallas import tpu as pltpu
from jax.experimental.pallas import tpu_sc as plsc

N = 100000
E = 50
L = 16
NC = 2
NS = 16
NW = NC * NS
G = 8
TILE_ROWS = G * L
NSLAB = E // 8

TC_ROWS = 77824
BC = 9728
TC_GRID = TC_ROWS // BC

CH = 640
SC_ROWS = N - TC_ROWS
REM = 32
EXTRA = (SC_ROWS - REM) // 128 - NW * (CH // 128)
TILES = CH // TILE_ROWS
SPAN_A = CH + 128
SPAN_B = CH
SPAN_LAST = CH + REM


def _sc_body(xt_hbm, ae_hbm, out_hbm, xbuf, tailbuf, aebuf, aexp, outbuf,
             sem0, sem1):
    wid = lax.axis_index("s") * NC + lax.axis_index("c")
    base_l = wid * CH + jnp.minimum(wid, EXTRA) * 128
    base = TC_ROWS + base_l
    zero = jnp.zeros((L,), jnp.float32)

    is_a = wid < EXTRA
    is_last = wid == NW - 1
    tail_row = pl.multiple_of(base + CH, 128)

    b128 = pl.multiple_of(base, 128)
    for q in range(CH // 128):
        pltpu.async_copy(xt_hbm.at[:, pl.ds(b128 + 128 * q, 128)],
                         xbuf.at[:, pl.ds(128 * q, 128)], sem0)
    pltpu.sync_copy(ae_hbm, aebuf.at[pl.ds(0, E)])
    for kc in range(0, E, L):
        chunk = aebuf[pl.ds(kc, L)]
        for k in range(kc, min(kc + L, E)):
            aexp[pl.ds(L * k, L)] = chunk[jnp.full((L,), k - kc, jnp.int32)]

    @pl.when(is_a)
    def _():
        pltpu.async_copy(xt_hbm.at[:, pl.ds(tail_row, 128)], tailbuf, sem1)

    @pl.when(is_last)
    def _():
        for s in range(NSLAB):
            pltpu.async_copy(xt_hbm.at[pl.ds(8 * s, 8), pl.ds(tail_row, REM)],
                             tailbuf.at[pl.ds(8 * s, 8), pl.ds(0, REM)], sem1)
        pltpu.async_copy(xt_hbm.at[pl.ds(8 * NSLAB, 2), pl.ds(tail_row, REM)],
                         tailbuf.at[pl.ds(8 * NSLAB, 2), pl.ds(0, REM)], sem1)

    for q in range(CH // 128):
        pltpu.make_async_copy(xt_hbm.at[:, pl.ds(0, 128)],
                              xbuf.at[:, pl.ds(128 * q, 128)], sem0).wait()

    @pl.loop(0, TILES)
    def _(t):
        cbase = t * TILE_ROWS
        accs = [zero] * G
        for k in range(E):
            aev = aexp[pl.ds(L * k, L)]
            for j in range(G):
                accs[j] = accs[j] + xbuf[k, pl.ds(cbase + L * j, L)] * aev
        for j in range(G):
            outbuf[pl.ds(cbase + L * j, L)] = accs[j]

    @pl.when(is_a)
    def _():
        pltpu.make_async_copy(xt_hbm.at[:, pl.ds(0, 128)], tailbuf,
                              sem1).wait()

    @pl.when(is_last)
    def _():
        for s in range(NSLAB):
            pltpu.make_async_copy(
                xt_hbm.at[pl.ds(0, 8), pl.ds(0, REM)],
                tailbuf.at[pl.ds(8 * s, 8), pl.ds(0, REM)], sem1).wait()
        pltpu.make_async_copy(
            xt_hbm.at[pl.ds(0, 2), pl.ds(0, REM)],
            tailbuf.at[pl.ds(8 * NSLAB, 2), pl.ds(0, REM)], sem1).wait()

    tail_n = jnp.where(is_a, 128, jnp.where(is_last, REM, 0))

    @pl.loop(0, tail_n // L)
    def _(g):
        acc0 = zero
        acc1 = zero
        for k in range(0, E, 2):
            aev0 = aexp[pl.ds(L * k, L)]
            aev1 = aexp[pl.ds(L * (k + 1), L)]
            acc0 = acc0 + tailbuf[k, pl.ds(g * L, L)] * aev0
            acc1 = acc1 + tailbuf[k + 1, pl.ds(g * L, L)] * aev1
        outbuf[pl.ds(CH + g * L, L)] = acc0 + acc1

    @pl.when(is_a)
    def _():
        pltpu.sync_copy(outbuf.at[pl.ds(0, SPAN_A)],
                        out_hbm.at[pl.ds(base_l, SPAN_A)])

    @pl.when(jnp.logical_and(jnp.logical_not(is_a), jnp.logical_not(is_last)))
    def _():
        pltpu.sync_copy(outbuf.at[pl.ds(0, SPAN_B)],
                        out_hbm.at[pl.ds(base_l, SPAN_B)])

    @pl.when(is_last)
    def _():
        pltpu.sync_copy(outbuf.at[pl.ds(0, SPAN_LAST)],
                        out_hbm.at[pl.ds(base_l, SPAN_LAST)])


@functools.partial(
    pl.kernel,
    out_type=jax.ShapeDtypeStruct((SC_ROWS,), jnp.float32),
    mesh=plsc.VectorSubcoreMesh(core_axis_name="c", subcore_axis_name="s"),
    compiler_params=pltpu.CompilerParams(
        needs_layout_passes=False, use_tc_tiling_on_sc=True),
    scratch_types=[
        pltpu.VMEM((E, CH), jnp.float32),
        pltpu.VMEM((E, 128), jnp.float32),
        pltpu.VMEM((64,), jnp.float32),
        pltpu.VMEM((E * L,), jnp.float32),
        pltpu.VMEM((SPAN_A,), jnp.float32),
        pltpu.SemaphoreType.DMA,
        pltpu.SemaphoreType.DMA,
    ],
)
def _sc_matvec(xt, ae, out_flat, xbuf, tailbuf, aebuf, aexp, outbuf,
               sem0, sem1):
    _sc_body(xt, ae, out_flat, xbuf, tailbuf, aebuf, aexp, outbuf, sem0, sem1)


def _tc_body(xt_ref, ae_ref, out_ref):
    out_ref[...] = jax.lax.dot_general(
        ae_ref[...], xt_ref[...],
        dimension_numbers=(((1,), (0,)), ((), ())))


_tc_matvec = pl.pallas_call(
    _tc_body,
    grid=(TC_GRID,),
    in_specs=[
        pl.BlockSpec((E, BC), lambda i: (0, i)),
        pl.BlockSpec((1, E), lambda i: (0, 0)),
    ],
    out_specs=pl.BlockSpec((1, BC), lambda i: (0, i)),
    out_shape=jax.ShapeDtypeStruct((1, TC_ROWS), jnp.float32),
)


@jax.jit
def kernel(x, atomic_energies):
    xt = x.T
    out_sc = _sc_matvec(xt, atomic_energies)
    out_tc = _tc_matvec(xt, atomic_energies[None, :])
    out = jnp.concatenate([out_tc[0], out_sc])
    return out[:, None]


# speedup vs baseline: 1.0914x; 1.0500x over previous
import functools
import jax
import jax.numpy as jnp
from jax import lax
from jax.experimental import pallas as pl
from jax.experimental.pallas import tpu as pltpu
from jax.experimental.pallas import tpu_sc as plsc

N = 100000
E = 50
L = 16
NC = 2
NS = 16
NW = NC * NS
G = 8
TILE_ROWS = G * L
NSLAB = E // 8

TC_ROWS = 83968
BC = 10496
TC_GRID = TC_ROWS // BC

CH = 384
SC_ROWS = N - TC_ROWS
REM = 32
EXTRA = (SC_ROWS - REM) // 128 - NW * (CH // 128)
TILES = CH // TILE_ROWS
SPAN_A = CH + 128
SPAN_B = CH
SPAN_LAST = CH + REM


def _sc_body(xt_hbm, ae_hbm, out_hbm, xbuf, tailbuf, aebuf, aexp, outbuf,
             sem0, sem1):
    wid = lax.axis_index("s") * NC + lax.axis_index("c")
    base_l = wid * CH + jnp.minimum(wid, EXTRA) * 128
    base = TC_ROWS + base_l
    zero = jnp.zeros((L,), jnp.float32)

    is_a = wid < EXTRA
    is_last = wid == NW - 1
    tail_row = pl.multiple_of(base + CH, 128)

    b128 = pl.multiple_of(base, 128)
    for q in range(CH // 128):
        pltpu.async_copy(xt_hbm.at[:, pl.ds(b128 + 128 * q, 128)],
                         xbuf.at[:, pl.ds(128 * q, 128)], sem0)
    pltpu.sync_copy(ae_hbm, aebuf.at[pl.ds(0, E)])
    for kc in range(0, E, L):
        chunk = aebuf[pl.ds(kc, L)]
        for k in range(kc, min(kc + L, E)):
            aexp[pl.ds(L * k, L)] = chunk[jnp.full((L,), k - kc, jnp.int32)]

    @pl.when(is_a)
    def _():
        pltpu.async_copy(xt_hbm.at[:, pl.ds(tail_row, 128)], tailbuf, sem1)

    @pl.when(is_last)
    def _():
        for s in range(NSLAB):
            pltpu.async_copy(xt_hbm.at[pl.ds(8 * s, 8), pl.ds(tail_row, REM)],
                             tailbuf.at[pl.ds(8 * s, 8), pl.ds(0, REM)], sem1)
        pltpu.async_copy(xt_hbm.at[pl.ds(8 * NSLAB, 2), pl.ds(tail_row, REM)],
                         tailbuf.at[pl.ds(8 * NSLAB, 2), pl.ds(0, REM)], sem1)

    for q in range(CH // 128):
        pltpu.make_async_copy(xt_hbm.at[:, pl.ds(0, 128)],
                              xbuf.at[:, pl.ds(128 * q, 128)], sem0).wait()

    @pl.loop(0, TILES)
    def _(t):
        cbase = t * TILE_ROWS
        accs = [zero] * G
        for k in range(E):
            aev = aexp[pl.ds(L * k, L)]
            for j in range(G):
                accs[j] = accs[j] + xbuf[k, pl.ds(cbase + L * j, L)] * aev
        for j in range(G):
            outbuf[pl.ds(cbase + L * j, L)] = accs[j]

    @pl.when(is_a)
    def _():
        pltpu.make_async_copy(xt_hbm.at[:, pl.ds(0, 128)], tailbuf,
                              sem1).wait()

    @pl.when(is_last)
    def _():
        for s in range(NSLAB):
            pltpu.make_async_copy(
                xt_hbm.at[pl.ds(0, 8), pl.ds(0, REM)],
                tailbuf.at[pl.ds(8 * s, 8), pl.ds(0, REM)], sem1).wait()
        pltpu.make_async_copy(
            xt_hbm.at[pl.ds(0, 2), pl.ds(0, REM)],
            tailbuf.at[pl.ds(8 * NSLAB, 2), pl.ds(0, REM)], sem1).wait()

    tail_n = jnp.where(is_a, 128, jnp.where(is_last, REM, 0))

    @pl.loop(0, tail_n // L)
    def _(g):
        acc0 = zero
        acc1 = zero
        for k in range(0, E, 2):
            aev0 = aexp[pl.ds(L * k, L)]
            aev1 = aexp[pl.ds(L * (k + 1), L)]
            acc0 = acc0 + tailbuf[k, pl.ds(g * L, L)] * aev0
            acc1 = acc1 + tailbuf[k + 1, pl.ds(g * L, L)] * aev1
        outbuf[pl.ds(CH + g * L, L)] = acc0 + acc1

    @pl.when(is_a)
    def _():
        pltpu.sync_copy(outbuf.at[pl.ds(0, SPAN_A)],
                        out_hbm.at[pl.ds(base_l, SPAN_A)])

    @pl.when(jnp.logical_and(jnp.logical_not(is_a), jnp.logical_not(is_last)))
    def _():
        pltpu.sync_copy(outbuf.at[pl.ds(0, SPAN_B)],
                        out_hbm.at[pl.ds(base_l, SPAN_B)])

    @pl.when(is_last)
    def _():
        pltpu.sync_copy(outbuf.at[pl.ds(0, SPAN_LAST)],
                        out_hbm.at[pl.ds(base_l, SPAN_LAST)])


@functools.partial(
    pl.kernel,
    out_type=jax.ShapeDtypeStruct((SC_ROWS,), jnp.float32),
    mesh=plsc.VectorSubcoreMesh(core_axis_name="c", subcore_axis_name="s"),
    compiler_params=pltpu.CompilerParams(
        needs_layout_passes=False, use_tc_tiling_on_sc=True),
    scratch_types=[
        pltpu.VMEM((E, CH), jnp.float32),
        pltpu.VMEM((E, 128), jnp.float32),
        pltpu.VMEM((64,), jnp.float32),
        pltpu.VMEM((E * L,), jnp.float32),
        pltpu.VMEM((SPAN_A,), jnp.float32),
        pltpu.SemaphoreType.DMA,
        pltpu.SemaphoreType.DMA,
    ],
)
def _sc_matvec(xt, ae, out_flat, xbuf, tailbuf, aebuf, aexp, outbuf,
               sem0, sem1):
    _sc_body(xt, ae, out_flat, xbuf, tailbuf, aebuf, aexp, outbuf, sem0, sem1)


def _tc_body(xt_ref, ae_ref, out_ref):
    out_ref[...] = jax.lax.dot_general(
        ae_ref[...], xt_ref[...],
        dimension_numbers=(((1,), (0,)), ((), ())))


_tc_matvec = pl.pallas_call(
    _tc_body,
    grid=(TC_GRID,),
    in_specs=[
        pl.BlockSpec((E, BC), lambda i: (0, i)),
        pl.BlockSpec((1, E), lambda i: (0, 0)),
    ],
    out_specs=pl.BlockSpec((1, BC), lambda i: (0, i)),
    out_shape=jax.ShapeDtypeStruct((1, TC_ROWS), jnp.float32),
)


@jax.jit
def kernel(x, atomic_energies):
    xt = x.T
    out_sc = _sc_matvec(xt, atomic_energies)
    out_tc = _tc_matvec(xt, atomic_energies[None, :])
    out = jnp.concatenate([out_tc[0], out_sc])
    return out[:, None]
